# initial kernel scaffold (unmeasured)
import jax
import jax.numpy as jnp
from jax import lax
from jax.experimental import pallas as pl
from jax.experimental.pallas import tpu as pltpu

N_DEV = 4


def kernel(table, idx):
    v_per, d = table.shape
    n = idx.shape[0]
    assert n % N_DEV == 0
    blk = n // N_DEV

    my = lax.axis_index("i")
    local_idx = idx - my * v_per
    own = (local_idx >= 0) & (local_idx < v_per)
    safe = jnp.where(own, local_idx, 0)
    partial = jnp.where(own[:, None], table[safe], jnp.float32(0))

    def body(p_ref, out_ref, comm, send_sems, recv_sems):
        my_pos = lax.axis_index("i")
        left = lax.rem(my_pos + N_DEV - 1, N_DEV)
        right = lax.rem(my_pos + 1, N_DEV)

        barrier_sem = pltpu.get_barrier_semaphore()
        for nbr in (left, right):
            pl.semaphore_signal(
                barrier_sem, inc=1,
                device_id=(nbr,), device_id_type=pl.DeviceIdType.MESH,
            )
        pl.semaphore_wait(barrier_sem, 2)

        out_ref[...] = p_ref[...]

        for s in range(N_DEV - 1):
            sb = lax.rem(my_pos - s + N_DEV, N_DEV)
            rb = lax.rem(my_pos - s - 1 + N_DEV, N_DEV)
            rdma = pltpu.make_async_remote_copy(
                src_ref=out_ref.at[pl.ds(sb * blk, blk), :],
                dst_ref=comm.at[s],
                send_sem=send_sems.at[s],
                recv_sem=recv_sems.at[s],
                device_id=(right,),
                device_id_type=pl.DeviceIdType.MESH,
            )
            rdma.start()
            rdma.wait()
            out_ref[pl.ds(rb * blk, blk), :] = (
                out_ref[pl.ds(rb * blk, blk), :] + comm[s]
            )

        for s in range(N_DEV - 1):
            sb = lax.rem(my_pos + 1 - s + N_DEV, N_DEV)
            rdma = pltpu.make_async_remote_copy(
                src_ref=out_ref.at[pl.ds(sb * blk, blk), :],
                dst_ref=out_ref.at[pl.ds(sb * blk, blk), :],
                send_sem=send_sems.at[N_DEV - 1 + s],
                recv_sem=recv_sems.at[N_DEV - 1 + s],
                device_id=(right,),
                device_id_type=pl.DeviceIdType.MESH,
            )
            rdma.start()
            rdma.wait()

    return pl.pallas_call(
        body,
        out_shape=jax.ShapeDtypeStruct((n, d), jnp.float32),
        in_specs=[pl.BlockSpec(memory_space=pltpu.VMEM)],
        out_specs=pl.BlockSpec(memory_space=pltpu.VMEM),
        scratch_shapes=[
            pltpu.VMEM((N_DEV - 1, blk, d), jnp.float32),
            pltpu.SemaphoreType.DMA((2 * (N_DEV - 1),)),
            pltpu.SemaphoreType.DMA((2 * (N_DEV - 1),)),
        ],
        compiler_params=pltpu.CompilerParams(collective_id=0),
    )(partial)


# baseline (device time: 194701 ns/iter reference)
import jax
import jax.numpy as jnp
from jax import lax
from jax.experimental import pallas as pl
from jax.experimental.pallas import tpu as pltpu

N_DEV = 4


def kernel(table, idx):
    v_per, d = table.shape
    n = idx.shape[0]
    assert n % N_DEV == 0
    blk = n // N_DEV

    def body(idx_ref, table_ref, out_ref, comm, send_sems, recv_sems):
        my_pos = lax.axis_index("i")
        left = lax.rem(my_pos + N_DEV - 1, N_DEV)
        right = lax.rem(my_pos + 1, N_DEV)

        barrier_sem = pltpu.get_barrier_semaphore()
        for nbr in (left, right):
            pl.semaphore_signal(
                barrier_sem, inc=1,
                device_id=(nbr,), device_id_type=pl.DeviceIdType.MESH,
            )
        pl.semaphore_wait(barrier_sem, 2)

        base = my_pos * v_per

        def gather_row(i, _):
            row = idx_ref[i] - base
            inb = (row >= 0) & (row < v_per)
            safe = jnp.where(inb, row, 0)
            vals = table_ref[pl.ds(safe, 1), :]
            out_ref[pl.ds(i, 1), :] = jnp.where(inb, vals, jnp.float32(0))
            return 0

        lax.fori_loop(0, n, gather_row, 0)

        for s in range(N_DEV - 1):
            sb = lax.rem(my_pos - s + N_DEV, N_DEV)
            rb = lax.rem(my_pos - s - 1 + N_DEV, N_DEV)
            rdma = pltpu.make_async_remote_copy(
                src_ref=out_ref.at[pl.ds(sb * blk, blk), :],
                dst_ref=comm.at[s],
                send_sem=send_sems.at[s],
                recv_sem=recv_sems.at[s],
                device_id=(right,),
                device_id_type=pl.DeviceIdType.MESH,
            )
            rdma.start()
            rdma.wait()
            out_ref[pl.ds(rb * blk, blk), :] = (
                out_ref[pl.ds(rb * blk, blk), :] + comm[s]
            )

        for s in range(N_DEV - 1):
            sb = lax.rem(my_pos + 1 - s + N_DEV, N_DEV)
            rdma = pltpu.make_async_remote_copy(
                src_ref=out_ref.at[pl.ds(sb * blk, blk), :],
                dst_ref=out_ref.at[pl.ds(sb * blk, blk), :],
                send_sem=send_sems.at[N_DEV - 1 + s],
                recv_sem=recv_sems.at[N_DEV - 1 + s],
                device_id=(right,),
                device_id_type=pl.DeviceIdType.MESH,
            )
            rdma.start()
            rdma.wait()

    return pl.pallas_call(
        body,
        out_shape=jax.ShapeDtypeStruct((n, d), jnp.float32),
        in_specs=[
            pl.BlockSpec(memory_space=pltpu.SMEM),
            pl.BlockSpec(memory_space=pltpu.VMEM),
        ],
        out_specs=pl.BlockSpec(memory_space=pltpu.VMEM),
        scratch_shapes=[
            pltpu.VMEM((N_DEV - 1, blk, d), jnp.float32),
            pltpu.SemaphoreType.DMA((2 * (N_DEV - 1),)),
            pltpu.SemaphoreType.DMA((2 * (N_DEV - 1),)),
        ],
        compiler_params=pltpu.CompilerParams(
            collective_id=0, vmem_limit_bytes=80 * 1024 * 1024
        ),
    )(idx, table)


# device time: 115818 ns/iter; 1.6811x vs baseline; 1.6811x over previous
import jax
import jax.numpy as jnp
from jax import lax
from jax.experimental import pallas as pl
from jax.experimental.pallas import tpu as pltpu

N_DEV = 4


def kernel(table, idx):
    v_per, d = table.shape
    n = idx.shape[0]
    assert n % (2 * N_DEV) == 0
    blk = n // N_DEV
    hblk = blk // 2

    idx2d = idx.reshape(n, 1)

    def body(idx_ref, idx2d_ref, table_ref, out_ref, comm,
             send_sems, recv_sems):
        my_pos = lax.axis_index("i")
        left = lax.rem(my_pos + N_DEV - 1, N_DEV)
        right = lax.rem(my_pos + 1, N_DEV)

        barrier_sem = pltpu.get_barrier_semaphore()
        for nbr in (left, right):
            pl.semaphore_signal(
                barrier_sem, inc=1,
                device_id=(nbr,), device_id_type=pl.DeviceIdType.MESH,
            )
        pl.semaphore_wait(barrier_sem, 2)

        base = my_pos * v_per

        def gather_block(bstart):
            def g(j, _):
                i = bstart + j
                row = idx_ref[i] - base

                @pl.when((row >= 0) & (row < v_per))
                def _():
                    out_ref[pl.ds(i, 1), :] = table_ref[pl.ds(row, 1), :]

                return 0

            lax.fori_loop(0, blk, g, 0)

        def own_mask(rstart):
            rows = idx2d_ref[pl.ds(rstart, hblk), :] - base
            return (rows >= 0) & (rows < v_per)

        def make_rs(s):
            sb_a = lax.rem(my_pos - s + N_DEV, N_DEV)
            sb_b = lax.rem(my_pos + s, N_DEV)
            a = pltpu.make_async_remote_copy(
                src_ref=out_ref.at[pl.ds(sb_a * blk, hblk), :],
                dst_ref=comm.at[0, s],
                send_sem=send_sems.at[s],
                recv_sem=recv_sems.at[s],
                device_id=(right,),
                device_id_type=pl.DeviceIdType.MESH,
            )
            b = pltpu.make_async_remote_copy(
                src_ref=out_ref.at[pl.ds(sb_b * blk + hblk, hblk), :],
                dst_ref=comm.at[1, s],
                send_sem=send_sems.at[2 * (N_DEV - 1) + s],
                recv_sem=recv_sems.at[2 * (N_DEV - 1) + s],
                device_id=(left,),
                device_id_type=pl.DeviceIdType.MESH,
            )
            return a, b

        def rs_finish(s, a, b):
            rb_a = lax.rem(my_pos - s - 1 + N_DEV, N_DEV)
            rb_b = lax.rem(my_pos + s + 1, N_DEV)
            a.wait()
            sa = rb_a * blk
            out_ref[pl.ds(sa, hblk), :] = jnp.where(
                own_mask(sa), out_ref[pl.ds(sa, hblk), :], comm[0, s]
            )
            b.wait()
            sb = rb_b * blk + hblk
            out_ref[pl.ds(sb, hblk), :] = jnp.where(
                own_mask(sb), out_ref[pl.ds(sb, hblk), :], comm[1, s]
            )

        gather_block(my_pos * blk)
        a0, b0 = make_rs(0)
        a0.start()
        b0.start()
        gather_block(lax.rem(my_pos + 1, N_DEV) * blk)
        gather_block(lax.rem(my_pos + 3, N_DEV) * blk)
        rs_finish(0, a0, b0)

        a1, b1 = make_rs(1)
        a1.start()
        b1.start()
        gather_block(lax.rem(my_pos + 2, N_DEV) * blk)
        rs_finish(1, a1, b1)

        a2, b2 = make_rs(2)
        a2.start()
        b2.start()
        rs_finish(2, a2, b2)

        for s in range(N_DEV - 1):
            sb_a = lax.rem(my_pos + 1 - s + N_DEV, N_DEV)
            sb_b = lax.rem(my_pos - 1 + s + N_DEV, N_DEV)
            a = pltpu.make_async_remote_copy(
                src_ref=out_ref.at[pl.ds(sb_a * blk, hblk), :],
                dst_ref=out_ref.at[pl.ds(sb_a * blk, hblk), :],
                send_sem=send_sems.at[N_DEV - 1 + s],
                recv_sem=recv_sems.at[N_DEV - 1 + s],
                device_id=(right,),
                device_id_type=pl.DeviceIdType.MESH,
            )
            b = pltpu.make_async_remote_copy(
                src_ref=out_ref.at[pl.ds(sb_b * blk + hblk, hblk), :],
                dst_ref=out_ref.at[pl.ds(sb_b * blk + hblk, hblk), :],
                send_sem=send_sems.at[3 * (N_DEV - 1) + s],
                recv_sem=recv_sems.at[3 * (N_DEV - 1) + s],
                device_id=(left,),
                device_id_type=pl.DeviceIdType.MESH,
            )
            a.start()
            b.start()
            a.wait()
            b.wait()

    return pl.pallas_call(
        body,
        out_shape=jax.ShapeDtypeStruct((n, d), jnp.float32),
        in_specs=[
            pl.BlockSpec(memory_space=pltpu.SMEM),
            pl.BlockSpec(memory_space=pltpu.VMEM),
            pl.BlockSpec(memory_space=pltpu.VMEM),
        ],
        out_specs=pl.BlockSpec(memory_space=pltpu.VMEM),
        scratch_shapes=[
            pltpu.VMEM((2, N_DEV - 1, hblk, d), jnp.float32),
            pltpu.SemaphoreType.DMA((4 * (N_DEV - 1),)),
            pltpu.SemaphoreType.DMA((4 * (N_DEV - 1),)),
        ],
        compiler_params=pltpu.CompilerParams(
            collective_id=0, vmem_limit_bytes=80 * 1024 * 1024
        ),
    )(idx, idx2d, table)


# device time: 97255 ns/iter; 2.0020x vs baseline; 1.1909x over previous
import jax
import jax.numpy as jnp
from jax import lax
from jax.experimental import pallas as pl
from jax.experimental.pallas import tpu as pltpu

N_DEV = 4
C = 2


def kernel(table, idx):
    v_per, d = table.shape
    n = idx.shape[0]
    assert n % (2 * C * N_DEV) == 0
    blk = n // N_DEV
    hblk = blk // 2
    ch = hblk // C
    R = n // ch

    def body(idx_ref, table_ref, out_ref, outi_ref, lrow_ref, cnt_ref,
             send_sems, recv_sems):
        my_pos = lax.axis_index("i")
        left = lax.rem(my_pos + N_DEV - 1, N_DEV)
        right = lax.rem(my_pos + 1, N_DEV)

        barrier_sem = pltpu.get_barrier_semaphore()
        for nbr in (left, right):
            pl.semaphore_signal(
                barrier_sem, inc=1,
                device_id=(nbr,), device_id_type=pl.DeviceIdType.MESH,
            )
        pl.semaphore_wait(barrier_sem, 2)

        base = my_pos * v_per

        def scan_merge(r):
            def g(j, _):
                i = r * ch + j
                row = idx_ref[i] - base

                @pl.when((row >= 0) & (row < v_per))
                def _():
                    out_ref[pl.ds(i, 1), :] = table_ref[pl.ds(row, 1), :]

                return 0

            lax.fori_loop(0, ch, g, 0)

        def pack(r):
            def g(j, cj):
                i = r * ch + j
                row = idx_ref[i] - base
                ok = (row >= 0) & (row < v_per)

                @pl.when(ok)
                def _():
                    outi_ref[r, cj] = i
                    lrow_ref[r, cj] = row

                return cj + jnp.where(ok, 1, 0)

            cnt_ref[r] = lax.fori_loop(0, ch, g, 0)

        def pmerge(r):
            def g(j, _):
                out_ref[pl.ds(outi_ref[r, j], 1), :] = table_ref[
                    pl.ds(lrow_ref[r, j], 1), :
                ]
                return 0

            lax.fori_loop(0, cnt_ref[r], g, 0)

        def copy(r, ring, s, c, dev):
            return pltpu.make_async_remote_copy(
                src_ref=out_ref.at[pl.ds(r * ch, ch), :],
                dst_ref=out_ref.at[pl.ds(r * ch, ch), :],
                send_sem=send_sems.at[ring, s, c],
                recv_sem=recv_sems.at[ring, s, c],
                device_id=(dev,),
                device_id_type=pl.DeviceIdType.MESH,
            )

        def r_a(b, c):
            return b * 2 * C + c

        def r_b(b, c):
            return b * 2 * C + C + c

        rs_a, rs_b, ag_a, ag_b = {}, {}, {}, {}

        for c in range(C):
            scan_merge(r_a(my_pos, c))
            rs_a[0, c] = copy(r_a(my_pos, c), 0, 0, c, right)
            rs_a[0, c].start()
            scan_merge(r_b(my_pos, c))
            rs_b[0, c] = copy(r_b(my_pos, c), 2, 0, c, left)
            rs_b[0, c].start()

        for s in range(1, N_DEV - 1):
            ba = lax.rem(my_pos - s + N_DEV, N_DEV)
            bb = lax.rem(my_pos + s, N_DEV)
            for c in range(C):
                pack(r_a(ba, c))
                pack(r_b(bb, c))
            for c in range(C):
                rs_a[s - 1, c].wait_recv()
                pmerge(r_a(ba, c))
                rs_a[s, c] = copy(r_a(ba, c), 0, s, c, right)
                rs_a[s, c].start()
                rs_b[s - 1, c].wait_recv()
                pmerge(r_b(bb, c))
                rs_b[s, c] = copy(r_b(bb, c), 2, s, c, left)
                rs_b[s, c].start()

        oa = lax.rem(my_pos + 1, N_DEV)
        ob = lax.rem(my_pos + 3, N_DEV)
        for c in range(C):
            pack(r_a(oa, c))
            pack(r_b(ob, c))
        for c in range(C):
            rs_a[N_DEV - 2, c].wait_recv()
            pmerge(r_a(oa, c))
            ag_a[0, c] = copy(r_a(oa, c), 1, 0, c, right)
            ag_a[0, c].start()
            rs_b[N_DEV - 2, c].wait_recv()
            pmerge(r_b(ob, c))
            ag_b[0, c] = copy(r_b(ob, c), 3, 0, c, left)
            ag_b[0, c].start()

        for s in range(1, N_DEV - 1):
            ba = lax.rem(my_pos + 1 - s + N_DEV, N_DEV)
            bb = lax.rem(my_pos - 1 + s + N_DEV, N_DEV)
            for c in range(C):
                ag_a[s - 1, c].wait_recv()
                ag_a[s, c] = copy(r_a(ba, c), 1, s, c, right)
                ag_a[s, c].start()
                ag_b[s - 1, c].wait_recv()
                ag_b[s, c] = copy(r_b(bb, c), 3, s, c, left)
                ag_b[s, c].start()

        for c in range(C):
            ag_a[N_DEV - 2, c].wait_recv()
            ag_b[N_DEV - 2, c].wait_recv()

        for dd in (rs_a, rs_b, ag_a, ag_b):
            for r in dd.values():
                r.wait_send()

    return pl.pallas_call(
        body,
        out_shape=jax.ShapeDtypeStruct((n, d), jnp.float32),
        in_specs=[
            pl.BlockSpec(memory_space=pltpu.SMEM),
            pl.BlockSpec(memory_space=pltpu.VMEM),
        ],
        out_specs=pl.BlockSpec(memory_space=pltpu.VMEM),
        scratch_shapes=[
            pltpu.SMEM((R, ch), jnp.int32),
            pltpu.SMEM((R, ch), jnp.int32),
            pltpu.SMEM((R,), jnp.int32),
            pltpu.SemaphoreType.DMA((4, N_DEV - 1, C)),
            pltpu.SemaphoreType.DMA((4, N_DEV - 1, C)),
        ],
        compiler_params=pltpu.CompilerParams(
            collective_id=0, vmem_limit_bytes=80 * 1024 * 1024
        ),
    )(idx, table)
